# Initial kernel scaffold; baseline (speedup 1.0000x reference)
import jax
import jax.numpy as jnp
from jax.experimental import pallas as pl


def kernel(t, spotlights, edge_index_initial, nodes_initial):
    raise NotImplementedError("write your pallas kernel here")


# speedup vs baseline: 26.4919x; 26.4919x over previous
import functools

import jax
import jax.numpy as jnp
from jax import lax
from jax.experimental import pallas as pl
from jax.experimental.pallas import tpu as pltpu
from jax.experimental.pallas import tpu_sc as plsc

N_NODES_ = 100000
DEG_PAD_ = 100352
DEG_TRASH_ = 100000
P_ = 4096
S_ = 128
OUT_DIM_ = 64
E2_ROWS_ = 25000
HIST_LOC_ = 2048 * 64
HIST_PAD_ = HIST_LOC_ + 512

_mesh = plsc.VectorSubcoreMesh(core_axis_name="c", subcore_axis_name="s")


@functools.partial(
    pl.kernel,
    out_type=jax.ShapeDtypeStruct((P_ * OUT_DIM_,), jnp.float32),
    mesh=_mesh,
    scratch_types=[
        pltpu.VMEM_SHARED((DEG_PAD_,), jnp.int32),
        pltpu.VMEM_SHARED((DEG_PAD_,), jnp.float32),
        pltpu.VMEM_SHARED((HIST_PAD_,), jnp.float32),
        pltpu.VMEM((8, 128), jnp.int32),
        pltpu.VMEM((128, 128), jnp.int32),
        pltpu.VMEM((128,), jnp.int32),
        pltpu.VMEM((128,), jnp.float32),
        pltpu.VMEM((1, 128), jnp.int32),
        pltpu.VMEM((128,), jnp.float32),
        pltpu.VMEM((128,), jnp.int32),
        pltpu.VMEM((6272,), jnp.int32),
        pltpu.VMEM((8224,), jnp.float32),
    ],
)
def _hist_kernel(edges_hbm, spot_hbm, nodes_hbm, out_hbm,
                 degs_sh, nodes_sh, hist_sh,
                 edge_buf, spot_buf, sd_buf, nval_buf,
                 hidx_buf, w_buf, ones_i, zero_i, zero_f):
    c = lax.axis_index("c")
    s = lax.axis_index("s")

    def _fill_zi(i, carry):
        zero_i[pl.ds(i * 16, 16)] = jnp.zeros((16,), jnp.int32)
        return carry
    lax.fori_loop(0, 392, _fill_zi, 0)

    def _fill_zf(i, carry):
        zero_f[pl.ds(i * 16, 16)] = jnp.zeros((16,), jnp.float32)
        return carry
    lax.fori_loop(0, 514, _fill_zf, 0)

    for k in range(8):
        ones_i[pl.ds(k * 16, 16)] = jnp.ones((16,), jnp.int32)

    pltpu.sync_copy(zero_i, degs_sh.at[pl.ds(s * 6272, 6272)])
    pltpu.sync_copy(zero_f.at[pl.ds(0, 8224)], hist_sh.at[pl.ds(s * 8224, 8224)])
    pltpu.sync_copy(nodes_hbm.at[pl.ds(s * 6272, 6272)],
                    nodes_sh.at[pl.ds(s * 6272, 6272)])
    plsc.subcore_barrier()

    start = s * 1568
    n_chunks = jnp.where(s < 15, 196, 185)

    def _p1(i, carry):
        off = start + i * 8
        pltpu.sync_copy(edges_hbm.at[pl.ds(off, 8)], edge_buf)
        for r in range(8):
            pltpu.sync_copy(ones_i, degs_sh.at[edge_buf.at[r]], add=True)
        return carry
    lax.fori_loop(0, n_chunks, _p1, 0)
    plsc.subcore_barrier()

    row0 = c * 2048 + s * 128
    pltpu.sync_copy(spot_hbm.at[pl.ds(row0, 128)], spot_buf)

    def _p2(j, carry):
        pltpu.sync_copy(degs_sh.at[spot_buf.at[j]], sd_buf)
        pltpu.sync_copy(nodes_sh.at[spot_buf.at[j]], nval_buf)
        hbase = (s * 128 + j) * OUT_DIM_
        for k in range(8):
            d = sd_buf[pl.ds(k * 16, 16)]
            nv = nval_buf[pl.ds(k * 16, 16)]
            idx = hbase + jnp.minimum(d, OUT_DIM_ - 1)
            w = jnp.where(d < OUT_DIM_, nv, jnp.zeros((16,), jnp.float32))
            hidx_buf[0, pl.ds(k * 16, 16)] = idx
            w_buf[pl.ds(k * 16, 16)] = w
        pltpu.sync_copy(w_buf, hist_sh.at[hidx_buf.at[0]], add=True)
        return carry
    lax.fori_loop(0, 128, _p2, 0)
    plsc.subcore_barrier()

    loc = s * (128 * OUT_DIM_)
    glob = c * HIST_LOC_ + loc
    pltpu.sync_copy(hist_sh.at[pl.ds(loc, 128 * OUT_DIM_)],
                    out_hbm.at[pl.ds(glob, 128 * OUT_DIM_)])


def kernel(t, spotlights, edge_index_initial, nodes_initial):
    del t
    spot = jnp.squeeze(spotlights, 0)
    edges = edge_index_initial.reshape(E2_ROWS_, 128)
    nodes = jnp.concatenate(
        [nodes_initial, jnp.zeros((DEG_PAD_ - N_NODES_,), jnp.float32)])
    out = _hist_kernel(edges, spot, nodes)
    return out.reshape(P_, OUT_DIM_)


# speedup vs baseline: 38.4642x; 1.4519x over previous
import functools

import jax
import jax.numpy as jnp
from jax import lax
from jax.experimental import pallas as pl
from jax.experimental.pallas import tpu as pltpu
from jax.experimental.pallas import tpu_sc as plsc

N_NODES_ = 100000
DEG_PAD_ = 100352
P_ = 4096
S_ = 128
OUT_DIM_ = 64
E2_ = 3200000
HIST_LOC_ = 2048 * 64
HIST_PAD_ = HIST_LOC_ + 512
CHUNK_ = 1024
FULL_CH_ = 196

_mesh = plsc.VectorSubcoreMesh(core_axis_name="c", subcore_axis_name="s")


@functools.partial(
    pl.kernel,
    out_type=jax.ShapeDtypeStruct((P_ * OUT_DIM_,), jnp.float32),
    mesh=_mesh,
    scratch_types=[
        pltpu.VMEM_SHARED((DEG_PAD_,), jnp.int32),
        pltpu.VMEM_SHARED((DEG_PAD_,), jnp.float32),
        pltpu.VMEM_SHARED((HIST_PAD_,), jnp.float32),
        pltpu.VMEM((CHUNK_,), jnp.int32),
        pltpu.VMEM((CHUNK_,), jnp.int32),
        pltpu.VMEM((16384,), jnp.int32),
        pltpu.VMEM((16384,), jnp.int32),
        pltpu.VMEM((16384,), jnp.float32),
        pltpu.VMEM((6272,), jnp.int32),
        pltpu.VMEM((8224,), jnp.float32),
    ],
)
def _hist_kernel(edges_hbm, spot_hbm, nodes_hbm, out_hbm,
                 degs_sh, nodes_sh, hist_sh,
                 edge_buf, ones_i, spot_idx, sd_buf, nval_buf,
                 zero_i, zero_f):
    c = lax.axis_index("c")
    s = lax.axis_index("s")

    def _fill_zi(i, carry):
        zero_i[pl.ds(i * 16, 16)] = jnp.zeros((16,), jnp.int32)
        return carry
    lax.fori_loop(0, 392, _fill_zi, 0)

    def _fill_zf(i, carry):
        zero_f[pl.ds(i * 16, 16)] = jnp.zeros((16,), jnp.float32)
        return carry
    lax.fori_loop(0, 514, _fill_zf, 0)

    def _fill_ones(i, carry):
        ones_i[pl.ds(i * 16, 16)] = jnp.ones((16,), jnp.int32)
        return carry
    lax.fori_loop(0, CHUNK_ // 16, _fill_ones, 0)

    pltpu.sync_copy(zero_i, degs_sh.at[pl.ds(s * 6272, 6272)])
    pltpu.sync_copy(zero_f, hist_sh.at[pl.ds(s * 8224, 8224)])
    pltpu.sync_copy(nodes_hbm.at[pl.ds(s * 6272, 6272)],
                    nodes_sh.at[pl.ds(s * 6272, 6272)])
    plsc.subcore_barrier()

    start = s * (FULL_CH_ * CHUNK_)
    n_chunks = jnp.where(s < 15, FULL_CH_, 185)

    def _p1(i, carry):
        off = start + i * CHUNK_
        pltpu.sync_copy(edges_hbm.at[pl.ds(off, CHUNK_)], edge_buf)
        pltpu.sync_copy(ones_i, degs_sh.at[edge_buf], add=True)
        return carry
    lax.fori_loop(0, n_chunks, _p1, 0)
    plsc.subcore_barrier()

    m0 = (c * 2048 + s * 128) * S_
    pltpu.sync_copy(spot_hbm.at[pl.ds(m0, 16384)], spot_idx)
    pltpu.sync_copy(degs_sh.at[spot_idx], sd_buf)
    pltpu.sync_copy(nodes_sh.at[spot_idx], nval_buf)

    def _p2(i, carry):
        hbase = (s * 128 + i // 8) * OUT_DIM_
        d = sd_buf[pl.ds(i * 16, 16)]
        nv = nval_buf[pl.ds(i * 16, 16)]
        idx = hbase + jnp.minimum(d, OUT_DIM_ - 1)
        w = jnp.where(d < OUT_DIM_, nv, jnp.zeros((16,), jnp.float32))
        sd_buf[pl.ds(i * 16, 16)] = idx
        nval_buf[pl.ds(i * 16, 16)] = w
        return carry
    lax.fori_loop(0, 1024, _p2, 0)
    pltpu.sync_copy(nval_buf, hist_sh.at[sd_buf], add=True)
    plsc.subcore_barrier()

    loc = s * (128 * OUT_DIM_)
    glob = c * HIST_LOC_ + loc
    pltpu.sync_copy(hist_sh.at[pl.ds(loc, 128 * OUT_DIM_)],
                    out_hbm.at[pl.ds(glob, 128 * OUT_DIM_)])


def kernel(t, spotlights, edge_index_initial, nodes_initial):
    del t
    spot = spotlights.reshape(P_ * S_)
    edges = edge_index_initial.reshape(E2_)
    nodes = jnp.concatenate(
        [nodes_initial, jnp.zeros((DEG_PAD_ - N_NODES_,), jnp.float32)])
    out = _hist_kernel(edges, spot, nodes)
    return out.reshape(P_, OUT_DIM_)


# speedup vs baseline: 71.4305x; 1.8571x over previous
import functools

import jax
import jax.numpy as jnp
from jax import lax
from jax.experimental import pallas as pl
from jax.experimental.pallas import tpu as pltpu
from jax.experimental.pallas import tpu_sc as plsc

N_NODES_ = 100000
DEG_PAD_ = 100352
P_ = 4096
S_ = 128
OUT_DIM_ = 64
E2_ = 3200000
HIST_LOC_ = 2048 * 64
HIST_PAD_ = HIST_LOC_ + 512
CHUNK_ = 8000
NCH_ = 25

_mesh = plsc.VectorSubcoreMesh(core_axis_name="c", subcore_axis_name="s")


@functools.partial(
    pl.kernel,
    out_type=jax.ShapeDtypeStruct((P_ * OUT_DIM_,), jnp.float32),
    mesh=_mesh,
    scratch_types=[
        pltpu.VMEM_SHARED((DEG_PAD_,), jnp.int32),
        pltpu.VMEM_SHARED((DEG_PAD_,), jnp.float32),
        pltpu.VMEM_SHARED((HIST_PAD_,), jnp.float32),
        pltpu.VMEM((CHUNK_,), jnp.int32),
        pltpu.VMEM((CHUNK_,), jnp.int32),
        pltpu.VMEM((CHUNK_,), jnp.int32),
        pltpu.VMEM((16384,), jnp.int32),
        pltpu.VMEM((16384,), jnp.int32),
        pltpu.VMEM((16384,), jnp.float32),
        pltpu.VMEM((6272,), jnp.int32),
        pltpu.VMEM((8224,), jnp.float32),
        pltpu.SemaphoreType.DMA,
        pltpu.SemaphoreType.DMA,
        pltpu.SemaphoreType.DMA,
        pltpu.SemaphoreType.DMA,
        pltpu.SemaphoreType.DMA,
    ],
)
def _hist_kernel(edges_hbm, spot_hbm, nodes_hbm, out_hbm,
                 degs_sh, nodes_sh, hist_sh,
                 eb0, eb1, ones_i, spot_idx, sd_buf, nval_buf,
                 zero_i, zero_f, sl0, sl1, ss0, ss1, sg):
    c = lax.axis_index("c")
    s = lax.axis_index("s")

    def _fill_zi(i, carry):
        zero_i[pl.ds(i * 16, 16)] = jnp.zeros((16,), jnp.int32)
        return carry
    lax.fori_loop(0, 392, _fill_zi, 0)

    def _fill_zf(i, carry):
        zero_f[pl.ds(i * 16, 16)] = jnp.zeros((16,), jnp.float32)
        return carry
    lax.fori_loop(0, 514, _fill_zf, 0)

    def _fill_ones(i, carry):
        ones_i[pl.ds(i * 16, 16)] = jnp.ones((16,), jnp.int32)
        return carry
    lax.fori_loop(0, CHUNK_ // 16, _fill_ones, 0)

    d1 = pltpu.async_copy(zero_i, degs_sh.at[pl.ds(s * 6272, 6272)], sl0)
    d2 = pltpu.async_copy(zero_f, hist_sh.at[pl.ds(s * 8224, 8224)], sl1)
    d3 = pltpu.async_copy(nodes_hbm.at[pl.ds(s * 6272, 6272)],
                          nodes_sh.at[pl.ds(s * 6272, 6272)], sg)
    d1.wait(); d2.wait(); d3.wait()
    plsc.subcore_barrier()

    start = s * (NCH_ * CHUNK_)
    bufs = (eb0, eb1)
    lsems = (sl0, sl1)
    ssems = (ss0, ss1)
    ld = [None, None]
    sc = [None, None]
    ld[0] = pltpu.async_copy(edges_hbm.at[pl.ds(start, CHUNK_)], eb0, sl0)
    for i in range(NCH_):
        b = i % 2
        o = 1 - b
        if i + 1 < NCH_:
            if sc[o] is not None:
                sc[o].wait()
            ld[o] = pltpu.async_copy(
                edges_hbm.at[pl.ds(start + (i + 1) * CHUNK_, CHUNK_)],
                bufs[o], lsems[o])
        ld[b].wait()
        sc[b] = pltpu.async_copy(ones_i, degs_sh.at[bufs[b]], ssems[b],
                                 add=True)
    sc[0].wait()
    sc[1].wait()
    plsc.subcore_barrier()

    m0 = (c * 2048 + s * 128) * S_
    pltpu.sync_copy(spot_hbm.at[pl.ds(m0, 16384)], spot_idx)
    g1 = pltpu.async_copy(degs_sh.at[spot_idx], sd_buf, sg)
    g2 = pltpu.async_copy(nodes_sh.at[spot_idx], nval_buf, ss0)
    g1.wait(); g2.wait()

    def _p2(j, carry):
        hbase = (s * 128 + j) * OUT_DIM_
        for k in range(8):
            off = j * 128 + k * 16
            d = sd_buf[pl.ds(off, 16)]
            nv = nval_buf[pl.ds(off, 16)]
            idx = hbase + jnp.minimum(d, OUT_DIM_ - 1)
            w = jnp.where(d < OUT_DIM_, nv, jnp.zeros((16,), jnp.float32))
            sd_buf[pl.ds(off, 16)] = idx
            nval_buf[pl.ds(off, 16)] = w
        return carry
    lax.fori_loop(0, 128, _p2, 0)
    pltpu.sync_copy(nval_buf, hist_sh.at[sd_buf], add=True)
    plsc.subcore_barrier()

    loc = s * (128 * OUT_DIM_)
    glob = c * HIST_LOC_ + loc
    pltpu.sync_copy(hist_sh.at[pl.ds(loc, 128 * OUT_DIM_)],
                    out_hbm.at[pl.ds(glob, 128 * OUT_DIM_)])


def kernel(t, spotlights, edge_index_initial, nodes_initial):
    del t
    spot = spotlights.reshape(P_ * S_)
    edges = edge_index_initial.reshape(E2_)
    nodes = jnp.concatenate(
        [nodes_initial, jnp.zeros((DEG_PAD_ - N_NODES_,), jnp.float32)])
    out = _hist_kernel(edges, spot, nodes)
    return out.reshape(P_, OUT_DIM_)


# speedup vs baseline: 74.4791x; 1.0427x over previous
import functools

import jax
import jax.numpy as jnp
from jax import lax
from jax.experimental import pallas as pl
from jax.experimental.pallas import tpu as pltpu
from jax.experimental.pallas import tpu_sc as plsc

N_NODES_ = 100000
DEG_PAD_ = 100352
P_ = 4096
S_ = 128
OUT_DIM_ = 64
E2_ = 3200000
HIST_LOC_ = 2048 * 64
HIST_PAD_ = HIST_LOC_ + 512
CHUNK_ = 8000
NCH_ = 25
HALF_ = 8192

_mesh = plsc.VectorSubcoreMesh(core_axis_name="c", subcore_axis_name="s")


@functools.partial(
    pl.kernel,
    out_type=jax.ShapeDtypeStruct((P_ * OUT_DIM_,), jnp.float32),
    mesh=_mesh,
    scratch_types=[
        pltpu.VMEM_SHARED((DEG_PAD_,), jnp.int32),
        pltpu.VMEM_SHARED((DEG_PAD_,), jnp.float32),
        pltpu.VMEM_SHARED((HIST_PAD_,), jnp.float32),
        pltpu.VMEM((CHUNK_,), jnp.int32),
        pltpu.VMEM((CHUNK_,), jnp.int32),
        pltpu.VMEM((CHUNK_,), jnp.int32),
        pltpu.VMEM((16384,), jnp.int32),
        pltpu.VMEM((HALF_,), jnp.int32),
        pltpu.VMEM((HALF_,), jnp.int32),
        pltpu.VMEM((HALF_,), jnp.float32),
        pltpu.VMEM((HALF_,), jnp.float32),
        pltpu.VMEM((6272,), jnp.int32),
        pltpu.VMEM((8224,), jnp.float32),
        pltpu.SemaphoreType.DMA,
        pltpu.SemaphoreType.DMA,
        pltpu.SemaphoreType.DMA,
        pltpu.SemaphoreType.DMA,
        pltpu.SemaphoreType.DMA,
        pltpu.SemaphoreType.DMA,
    ],
)
def _hist_kernel(edges_hbm, spot_hbm, nodes_hbm, out_hbm,
                 degs_sh, nodes_sh, hist_sh,
                 eb0, eb1, ones_i, spot_idx, sd0, sd1, nv0, nv1,
                 zero_i, zero_f, sa, sb, sc_, sd_, se, sf):
    c = lax.axis_index("c")
    s = lax.axis_index("s")

    def _fill_zi(i, carry):
        for u in range(4):
            zero_i[pl.ds(i * 64 + u * 16, 16)] = jnp.zeros((16,), jnp.int32)
        return carry
    lax.fori_loop(0, 98, _fill_zi, 0)

    def _fill_zf(i, carry):
        for u in range(4):
            zero_f[pl.ds(i * 64 + u * 16, 16)] = jnp.zeros((16,), jnp.float32)
        return carry
    lax.fori_loop(0, 128, _fill_zf, 0)
    for u in range(2):
        zero_f[pl.ds(8192 + u * 16, 16)] = jnp.zeros((16,), jnp.float32)

    def _fill_ones(i, carry):
        for u in range(4):
            ones_i[pl.ds(i * 64 + u * 16, 16)] = jnp.ones((16,), jnp.int32)
        return carry
    lax.fori_loop(0, 125, _fill_ones, 0)

    d1 = pltpu.async_copy(zero_i, degs_sh.at[pl.ds(s * 6272, 6272)], sa)
    d2 = pltpu.async_copy(zero_f, hist_sh.at[pl.ds(s * 8224, 8224)], sb)
    d3 = pltpu.async_copy(nodes_hbm.at[pl.ds(s * 6272, 6272)],
                          nodes_sh.at[pl.ds(s * 6272, 6272)], sc_)
    d1.wait(); d2.wait(); d3.wait()
    plsc.subcore_barrier()

    m0 = (c * 2048 + s * 128) * S_
    dspot = pltpu.async_copy(spot_hbm.at[pl.ds(m0, 16384)], spot_idx, se)

    start = s * (NCH_ * CHUNK_)
    bufs = (eb0, eb1)
    lsems = (sa, sb)
    ssems = (sc_, sd_)
    ld = [None, None]
    sc = [None, None]
    ld[0] = pltpu.async_copy(edges_hbm.at[pl.ds(start, CHUNK_)], eb0, sa)
    gnv = [None, None]
    for i in range(NCH_):
        b = i % 2
        o = 1 - b
        if i + 1 < NCH_:
            if sc[o] is not None:
                sc[o].wait()
            ld[o] = pltpu.async_copy(
                edges_hbm.at[pl.ds(start + (i + 1) * CHUNK_, CHUNK_)],
                bufs[o], lsems[o])
        ld[b].wait()
        sc[b] = pltpu.async_copy(ones_i, degs_sh.at[bufs[b]], ssems[b],
                                 add=True)
        if i == 0:
            dspot.wait()
            gnv[0] = pltpu.async_copy(
                nodes_sh.at[spot_idx.at[pl.ds(0, HALF_)]], nv0, se)
            gnv[1] = pltpu.async_copy(
                nodes_sh.at[spot_idx.at[pl.ds(HALF_, HALF_)]], nv1, sf)
    sc[0].wait()
    sc[1].wait()
    gnv[0].wait()
    gnv[1].wait()
    plsc.subcore_barrier()

    g0 = pltpu.async_copy(degs_sh.at[spot_idx.at[pl.ds(0, HALF_)]], sd0, sa)
    g1 = pltpu.async_copy(degs_sh.at[spot_idx.at[pl.ds(HALF_, HALF_)]],
                          sd1, sb)

    def _compute_half(h, sdb, nvb):
        def _p2(j, carry):
            hbase = (s * 128 + h * 64 + j) * OUT_DIM_
            for k in range(8):
                off = j * 128 + k * 16
                d = sdb[pl.ds(off, 16)]
                nv = nvb[pl.ds(off, 16)]
                idx = hbase + jnp.minimum(d, OUT_DIM_ - 1)
                w = jnp.where(d < OUT_DIM_, nv, jnp.zeros((16,), jnp.float32))
                sdb[pl.ds(off, 16)] = idx
                nvb[pl.ds(off, 16)] = w
            return carry
        lax.fori_loop(0, 64, _p2, 0)

    g0.wait()
    _compute_half(0, sd0, nv0)
    h0 = pltpu.async_copy(nv0, hist_sh.at[sd0], sc_, add=True)
    g1.wait()
    _compute_half(1, sd1, nv1)
    h1 = pltpu.async_copy(nv1, hist_sh.at[sd1], sd_, add=True)
    h0.wait()
    h1.wait()
    plsc.subcore_barrier()

    loc = s * (128 * OUT_DIM_)
    glob = c * HIST_LOC_ + loc
    pltpu.sync_copy(hist_sh.at[pl.ds(loc, 128 * OUT_DIM_)],
                    out_hbm.at[pl.ds(glob, 128 * OUT_DIM_)])


def kernel(t, spotlights, edge_index_initial, nodes_initial):
    del t
    spot = spotlights.reshape(P_ * S_)
    edges = edge_index_initial.reshape(E2_)
    nodes = jnp.concatenate(
        [nodes_initial, jnp.zeros((DEG_PAD_ - N_NODES_,), jnp.float32)])
    out = _hist_kernel(edges, spot, nodes)
    return out.reshape(P_, OUT_DIM_)


# speedup vs baseline: 90.8507x; 1.2198x over previous
import functools

import jax
import jax.numpy as jnp
from jax import lax
from jax.experimental import pallas as pl
from jax.experimental.pallas import tpu as pltpu
from jax.experimental.pallas import tpu_sc as plsc

N_NODES_ = 100000
DEG_PAD_ = 100352
DEG_TRASH_ = 100000
P_ = 4096
S_ = 128
OUT_DIM_ = 64
N_EDGES_ = 1600000
HIST_LOC_ = 2048 * 64
HIST_PAD_ = HIST_LOC_ + 512
CW_ = 1280
NCH_ = 78
TSPAN_ = NCH_ * CW_
HALF_ = 8192

_mesh = plsc.VectorSubcoreMesh(core_axis_name="c", subcore_axis_name="s")


@functools.partial(
    pl.kernel,
    out_type=jax.ShapeDtypeStruct((P_ * OUT_DIM_,), jnp.float32),
    mesh=_mesh,
    scratch_types=[
        pltpu.VMEM_SHARED((DEG_PAD_,), jnp.int32),
        pltpu.VMEM_SHARED((DEG_PAD_,), jnp.float32),
        pltpu.VMEM_SHARED((HIST_PAD_,), jnp.float32),
        pltpu.VMEM((2, CW_), jnp.int32),
        pltpu.VMEM((2, CW_), jnp.int32),
        pltpu.VMEM((2 * CW_,), jnp.int32),
        pltpu.VMEM((2 * CW_,), jnp.int32),
        pltpu.VMEM((2, 256), jnp.int32),
        pltpu.VMEM((512,), jnp.int32),
        pltpu.VMEM((2 * CW_,), jnp.int32),
        pltpu.VMEM((16384,), jnp.int32),
        pltpu.VMEM((HALF_,), jnp.int32),
        pltpu.VMEM((HALF_,), jnp.int32),
        pltpu.VMEM((HALF_,), jnp.float32),
        pltpu.VMEM((HALF_,), jnp.float32),
        pltpu.VMEM((6272,), jnp.int32),
        pltpu.VMEM((8224,), jnp.float32),
        pltpu.SemaphoreType.DMA,
        pltpu.SemaphoreType.DMA,
        pltpu.SemaphoreType.DMA,
        pltpu.SemaphoreType.DMA,
        pltpu.SemaphoreType.DMA,
        pltpu.SemaphoreType.DMA,
    ],
)
def _hist_kernel(edges_hbm, spot_hbm, nodes_hbm, out_hbm,
                 degs_sh, nodes_sh, hist_sh,
                 eb0, eb1, x0, x1, etail, xtail, ones_i, spot_idx,
                 sd0, sd1, nv0, nv1, zero_i, zero_f,
                 la, lb, sa, sb, se, sf):
    c = lax.axis_index("c")
    s = lax.axis_index("s")

    def _fill_zi(i, carry):
        for u in range(4):
            zero_i[pl.ds(i * 64 + u * 16, 16)] = jnp.zeros((16,), jnp.int32)
        return carry
    lax.fori_loop(0, 98, _fill_zi, 0)

    def _fill_zf(i, carry):
        for u in range(4):
            zero_f[pl.ds(i * 64 + u * 16, 16)] = jnp.zeros((16,), jnp.float32)
        return carry
    lax.fori_loop(0, 128, _fill_zf, 0)
    for u in range(2):
        zero_f[pl.ds(8192 + u * 16, 16)] = jnp.zeros((16,), jnp.float32)

    def _fill_ones(i, carry):
        for u in range(4):
            ones_i[pl.ds(i * 64 + u * 16, 16)] = jnp.ones((16,), jnp.int32)
        return carry
    lax.fori_loop(0, 2 * CW_ // 64, _fill_ones, 0)

    d1 = pltpu.async_copy(zero_i, degs_sh.at[pl.ds(s * 6272, 6272)], la)
    d2 = pltpu.async_copy(zero_f, hist_sh.at[pl.ds(s * 8224, 8224)], lb)
    d3 = pltpu.async_copy(nodes_hbm.at[pl.ds(s * 6272, 6272)],
                          nodes_sh.at[pl.ds(s * 6272, 6272)], se)
    d1.wait(); d2.wait(); d3.wait()
    plsc.subcore_barrier()

    m0 = (c * 2048 + s * 128) * S_
    dspot = pltpu.async_copy(spot_hbm.at[pl.ds(m0, 16384)], spot_idx, se)

    col0 = pl.multiple_of(s * TSPAN_, 128)

    def _load(chunk, buf, sem):
        off = pl.multiple_of(col0 + chunk * CW_, 128)
        return pltpu.async_copy(edges_hbm.at[:, pl.ds(off, CW_)], buf, sem)

    def _repack(buf, xb):
        def _rp(j, carry):
            for r in range(2):
                for u in range(4):
                    v = buf[r, pl.ds(j * 64 + u * 16, 16)]
                    xb[pl.ds(r * CW_ + j * 64 + u * 16, 16)] = v
            return carry
        lax.fori_loop(0, CW_ // 64, _rp, 0)

    def _fire(xb, sem):
        return pltpu.async_copy(ones_i, degs_sh.at[xb], sem, add=True)

    def _drain_scatter(xb, sem):
        pltpu.make_async_copy(ones_i, degs_sh.at[xb], sem).wait()

    _load(0, eb0, la).wait()
    _load(1, eb1, lb)
    _repack(eb0, x0)
    _load(2, eb0, la)
    _fire(x0, sa)
    dspot.wait()
    pltpu.async_copy(nodes_sh.at[spot_idx.at[pl.ds(0, HALF_)]], nv0, se)
    pltpu.async_copy(nodes_sh.at[spot_idx.at[pl.ds(HALF_, HALF_)]], nv1, sf)
    pltpu.make_async_copy(edges_hbm.at[:, pl.ds(0, CW_)], eb1, lb).wait()
    _repack(eb1, x1)
    _load(3, eb1, lb)
    _fire(x1, sb)

    def _ring(i, carry):
        c0 = 2 * i
        pltpu.make_async_copy(edges_hbm.at[:, pl.ds(0, CW_)], eb0, la).wait()
        _drain_scatter(x0, sa)
        _repack(eb0, x0)
        _load(c0 + 2, eb0, la)
        _fire(x0, sa)
        pltpu.make_async_copy(edges_hbm.at[:, pl.ds(0, CW_)], eb1, lb).wait()
        _drain_scatter(x1, sb)
        _repack(eb1, x1)
        _load(c0 + 3, eb1, lb)
        _fire(x1, sb)
        return carry
    lax.fori_loop(1, NCH_ // 2, _ring, 0)
    pltpu.make_async_copy(edges_hbm.at[:, pl.ds(0, CW_)], eb0, la).wait()
    pltpu.make_async_copy(edges_hbm.at[:, pl.ds(0, CW_)], eb1, lb).wait()
    _drain_scatter(x0, sa)
    _drain_scatter(x1, sb)

    toff = pl.multiple_of(
        jnp.where(s < 4, (12480 + 2 * s) * 128, (12483 + s) * 128), 128)
    pltpu.sync_copy(edges_hbm.at[:, pl.ds(toff, 256)], etail)
    keep_all = s < 4
    for r in range(2):
        for j in range(16):
            v = etail[r, pl.ds(j * 16, 16)]
            if j < 8:
                v = jnp.where(keep_all, v,
                              jnp.full((16,), DEG_TRASH_, jnp.int32))
            xtail[pl.ds(r * 256 + j * 16, 16)] = v
    tdone = pltpu.async_copy(ones_i.at[pl.ds(0, 512)], degs_sh.at[xtail],
                             sa, add=True)
    tdone.wait()
    pltpu.make_async_copy(nodes_sh.at[spot_idx.at[pl.ds(0, HALF_)]],
                          nv0, se).wait()
    pltpu.make_async_copy(nodes_sh.at[spot_idx.at[pl.ds(HALF_, HALF_)]],
                          nv1, sf).wait()
    plsc.subcore_barrier()

    g0 = pltpu.async_copy(degs_sh.at[spot_idx.at[pl.ds(0, HALF_)]], sd0, la)
    g1 = pltpu.async_copy(degs_sh.at[spot_idx.at[pl.ds(HALF_, HALF_)]],
                          sd1, lb)

    def _compute_half(h, sdb, nvb):
        def _p2(j, carry):
            hbase = (s * 128 + h * 64 + j) * OUT_DIM_
            for k in range(8):
                off = j * 128 + k * 16
                d = sdb[pl.ds(off, 16)]
                nv = nvb[pl.ds(off, 16)]
                idx = hbase + jnp.minimum(d, OUT_DIM_ - 1)
                w = jnp.where(d < OUT_DIM_, nv, jnp.zeros((16,), jnp.float32))
                sdb[pl.ds(off, 16)] = idx
                nvb[pl.ds(off, 16)] = w
            return carry
        lax.fori_loop(0, 64, _p2, 0)

    g0.wait()
    _compute_half(0, sd0, nv0)
    h0 = pltpu.async_copy(nv0, hist_sh.at[sd0], sa, add=True)
    g1.wait()
    _compute_half(1, sd1, nv1)
    h1 = pltpu.async_copy(nv1, hist_sh.at[sd1], sb, add=True)
    h0.wait()
    h1.wait()
    plsc.subcore_barrier()

    loc = s * (128 * OUT_DIM_)
    glob = c * HIST_LOC_ + loc
    pltpu.sync_copy(hist_sh.at[pl.ds(loc, 128 * OUT_DIM_)],
                    out_hbm.at[pl.ds(glob, 128 * OUT_DIM_)])


def kernel(t, spotlights, edge_index_initial, nodes_initial):
    del t
    spot = spotlights.reshape(P_ * S_)
    nodes = jnp.concatenate(
        [nodes_initial, jnp.zeros((DEG_PAD_ - N_NODES_,), jnp.float32)])
    out = _hist_kernel(edge_index_initial, spot, nodes)
    return out.reshape(P_, OUT_DIM_)
